# Initial kernel scaffold; baseline (speedup 1.0000x reference)
import jax
import jax.numpy as jnp
from jax.experimental import pallas as pl


def kernel(features, conv_w, conv_b, cls_w, cls_b, bbox_w, bbox_b):
    raise NotImplementedError("write your pallas kernel here")


# speedup vs baseline: 11.9431x; 11.9431x over previous
import numpy as np
import jax
import jax.numpy as jnp
from jax import lax
from jax.experimental import pallas as pl

H, W = 100, 152
STRIDE = 16
SIZES = (32.0, 64.0, 128.0, 256.0, 512.0)
RATIOS = (0.5, 1.0, 2.0)
NA = len(SIZES) * len(RATIOS)
PRE_NMS = 6000
POST_NMS = 1000
NMS_THRESH = 0.7
BBOX_CLIP = float(np.log(1000.0 / 16.0))
IM_W = W * STRIDE
IM_H = H * STRIDE

HWP = 15360
COL_T = 512
N_COL_T = HWP // COL_T
NPAD = 6144
NR, NC = 48, 128


def _anchors_np():
    base = []
    for size in SIZES:
        area = size * size
        for r in RATIOS:
            w = np.sqrt(area / r)
            h = w * r
            base.append([-w / 2.0, -h / 2.0, w / 2.0, h / 2.0])
    base = np.asarray(base, np.float32)
    sx = np.arange(W, dtype=np.float32) * STRIDE
    sy = np.arange(H, dtype=np.float32) * STRIDE
    yy, xx = np.meshgrid(sy, sx, indexing="ij")
    shifts = np.stack([xx, yy, xx, yy], axis=-1).reshape(-1, 1, 4)
    return (shifts + base[None]).reshape(-1, 4).astype(np.float32)


_ANCHORS = jnp.asarray(_anchors_np())


def _head_body(p_ref, w1_ref, b1_ref, w2_ref, b2_ref, o_ref):
    t = jnp.dot(w1_ref[...], p_ref[...], preferred_element_type=jnp.float32)
    t = jnp.maximum(t + b1_ref[...], 0.0)
    o_ref[...] = (
        jnp.dot(w2_ref[...], t, preferred_element_type=jnp.float32) + b2_ref[...]
    )


def _nms_body(d_ref, a_ref, box_ref, keep_ref):
    ax1, ay1, ax2, ay2 = a_ref[0], a_ref[1], a_ref[2], a_ref[3]
    widths = ax2 - ax1 + 1.0
    heights = ay2 - ay1 + 1.0
    cx = ax1 + 0.5 * widths
    cy = ay1 + 0.5 * heights
    dx, dy = d_ref[0], d_ref[1]
    dw = jnp.minimum(d_ref[2], BBOX_CLIP)
    dh = jnp.minimum(d_ref[3], BBOX_CLIP)
    pcx = dx * widths + cx
    pcy = dy * heights + cy
    pw = jnp.exp(dw) * widths
    ph = jnp.exp(dh) * heights
    x1 = jnp.clip(pcx - 0.5 * pw, 0.0, IM_W - 1.0)
    y1 = jnp.clip(pcy - 0.5 * ph, 0.0, IM_H - 1.0)
    x2 = jnp.clip(pcx + 0.5 * pw - 1.0, 0.0, IM_W - 1.0)
    y2 = jnp.clip(pcy + 0.5 * ph - 1.0, 0.0, IM_H - 1.0)
    box_ref[0], box_ref[1], box_ref[2], box_ref[3] = x1, y1, x2, y2
    area = (x2 - x1 + 1.0) * (y2 - y1 + 1.0)

    rid = lax.broadcasted_iota(jnp.int32, (NR, NC), 0)
    cid = lax.broadcasted_iota(jnp.int32, (NR, NC), 1)
    ii = rid * NC + cid

    def body(i, sup):
        eq = ii == i
        s_i = jnp.max(jnp.where(eq, sup, 0.0))
        bx1 = jnp.sum(jnp.where(eq, x1, 0.0))
        by1 = jnp.sum(jnp.where(eq, y1, 0.0))
        bx2 = jnp.sum(jnp.where(eq, x2, 0.0))
        by2 = jnp.sum(jnp.where(eq, y2, 0.0))
        ba = jnp.sum(jnp.where(eq, area, 0.0))
        iw = jnp.maximum(jnp.minimum(x2, bx2) - jnp.maximum(x1, bx1) + 1.0, 0.0)
        ih = jnp.maximum(jnp.minimum(y2, by2) - jnp.maximum(y1, by1) + 1.0, 0.0)
        inter = iw * ih
        iou = inter / (area + ba - inter)
        hit = jnp.where((iou > NMS_THRESH) & (ii > i), 1.0 - s_i, 0.0)
        return jnp.maximum(sup, hit)

    sup = lax.fori_loop(0, PRE_NMS, body, jnp.zeros((NR, NC), jnp.float32))
    keep_ref[...] = 1.0 - sup


def kernel(features, conv_w, conv_b, cls_w, cls_b, bbox_w, bbox_b):
    x = features[0]
    xp = jnp.pad(x, ((0, 0), (1, 1), (1, 1)))
    pat = jnp.stack(
        [xp[:, ky : ky + H, kx : kx + W] for ky in range(3) for kx in range(3)],
        axis=1,
    ).reshape(256 * 9, H * W)
    pat = jnp.pad(pat, ((0, 0), (0, HWP - H * W)))

    w1 = conv_w.reshape(256, 256 * 9)
    w2 = jnp.concatenate(
        [
            cls_w.reshape(NA, 256),
            bbox_w.reshape(4 * NA, 256),
            jnp.zeros((128 - 5 * NA, 256), jnp.float32),
        ],
        axis=0,
    )
    b2 = jnp.concatenate(
        [cls_b, bbox_b, jnp.zeros((128 - 5 * NA,), jnp.float32)], axis=0
    )

    head_out = pl.pallas_call(
        _head_body,
        grid=(N_COL_T,),
        in_specs=[
            pl.BlockSpec((256 * 9, COL_T), lambda j: (0, j)),
            pl.BlockSpec((256, 256 * 9), lambda j: (0, 0)),
            pl.BlockSpec((256, 1), lambda j: (0, 0)),
            pl.BlockSpec((128, 256), lambda j: (0, 0)),
            pl.BlockSpec((128, 1), lambda j: (0, 0)),
        ],
        out_specs=pl.BlockSpec((128, COL_T), lambda j: (0, j)),
        out_shape=jax.ShapeDtypeStruct((128, HWP), jnp.float32),
    )(pat, w1, conv_b[:, None], w2, b2[:, None])

    head_out = head_out[:, : H * W]
    obj = head_out[:NA].T.reshape(-1)
    reg = head_out[NA : 5 * NA].T.reshape(-1, 4)

    topv, topi = lax.top_k(obj, PRE_NMS)
    deltas = jnp.pad(reg[topi], ((0, NPAD - PRE_NMS), (0, 0)))
    anc = jnp.pad(_ANCHORS[topi], ((0, NPAD - PRE_NMS), (0, 0)))

    boxes, keep = pl.pallas_call(
        _nms_body,
        in_specs=[
            pl.BlockSpec((4, NR, NC), lambda: (0, 0, 0)),
            pl.BlockSpec((4, NR, NC), lambda: (0, 0, 0)),
        ],
        out_specs=[
            pl.BlockSpec((4, NR, NC), lambda: (0, 0, 0)),
            pl.BlockSpec((NR, NC), lambda: (0, 0)),
        ],
        out_shape=[
            jax.ShapeDtypeStruct((4, NR, NC), jnp.float32),
            jax.ShapeDtypeStruct((NR, NC), jnp.float32),
        ],
    )(deltas.T.reshape(4, NR, NC), anc.T.reshape(4, NR, NC))

    boxes = boxes.reshape(4, NPAD).T[:PRE_NMS]
    keep = keep.reshape(NPAD)[:PRE_NMS] > 0.5
    scores = jax.nn.sigmoid(topv)
    masked = jnp.where(keep, scores, -1.0)
    selv, seli = lax.top_k(masked, POST_NMS)
    return jnp.concatenate([boxes[seli], selv[:, None]], axis=1)


# speedup vs baseline: 17.1522x; 1.4362x over previous
import numpy as np
import jax
import jax.numpy as jnp
from jax import lax
from jax.experimental import pallas as pl

H, W = 100, 152
STRIDE = 16
SIZES = (32.0, 64.0, 128.0, 256.0, 512.0)
RATIOS = (0.5, 1.0, 2.0)
NA = len(SIZES) * len(RATIOS)
PRE_NMS = 6000
POST_NMS = 1000
NMS_THRESH = 0.7
BBOX_CLIP = float(np.log(1000.0 / 16.0))
IM_W = W * STRIDE
IM_H = H * STRIDE

HWP = 15360
COL_T = 512
N_COL_T = HWP // COL_T
NPAD = 6144
NR, NC = 48, 128


def _anchors_np():
    base = []
    for size in SIZES:
        area = size * size
        for r in RATIOS:
            w = np.sqrt(area / r)
            h = w * r
            base.append([-w / 2.0, -h / 2.0, w / 2.0, h / 2.0])
    base = np.asarray(base, np.float32)
    sx = np.arange(W, dtype=np.float32) * STRIDE
    sy = np.arange(H, dtype=np.float32) * STRIDE
    yy, xx = np.meshgrid(sy, sx, indexing="ij")
    shifts = np.stack([xx, yy, xx, yy], axis=-1).reshape(-1, 1, 4)
    return (shifts + base[None]).reshape(-1, 4).astype(np.float32)


_ANCHORS = jnp.asarray(_anchors_np())


def _head_body(p_ref, w1_ref, b1_ref, w2_ref, b2_ref, o_ref):
    t = jnp.dot(w1_ref[...], p_ref[...], preferred_element_type=jnp.float32)
    t = jnp.maximum(t + b1_ref[...], 0.0)
    o_ref[...] = (
        jnp.dot(w2_ref[...], t, preferred_element_type=jnp.float32) + b2_ref[...]
    )


def _nms_body(c_ref, rows_ref, keep_ref):
    x1, y1, x2, y2, area = c_ref[0], c_ref[1], c_ref[2], c_ref[3], c_ref[4]
    rid = lax.broadcasted_iota(jnp.int32, (NR, NC), 0)
    cid = lax.broadcasted_iota(jnp.int32, (NR, NC), 1)
    ii = rid * NC + cid

    def cond_fn(carry):
        i, kept, _ = carry
        return (i < PRE_NMS) & (kept < POST_NMS)

    def body_fn(carry):
        i, kept, sup = carry
        eq = ii == i
        s_i = jnp.max(jnp.where(eq, sup, 0.0))

        def live(s):
            row = rows_ref[pl.ds(i, 1), :]
            bx1, by1 = row[:, 0:1], row[:, 1:2]
            bx2, by2 = row[:, 2:3], row[:, 3:4]
            ba = row[:, 4:5]
            iw = jnp.maximum(
                jnp.minimum(x2, bx2) - jnp.maximum(x1, bx1) + 1.0, 0.0
            )
            ih = jnp.maximum(
                jnp.minimum(y2, by2) - jnp.maximum(y1, by1) + 1.0, 0.0
            )
            inter = iw * ih
            iou = inter / (area + ba - inter)
            hit = jnp.where((iou > NMS_THRESH) & (ii > i), 1.0, 0.0)
            return jnp.maximum(s, hit)

        sup2 = lax.cond(s_i < 0.5, live, lambda s: s, sup)
        return i + 1, kept + (1 - s_i.astype(jnp.int32)), sup2

    _, _, sup = lax.while_loop(
        cond_fn, body_fn, (0, 0, jnp.zeros((NR, NC), jnp.float32))
    )
    keep_ref[...] = 1.0 - sup


def kernel(features, conv_w, conv_b, cls_w, cls_b, bbox_w, bbox_b):
    x = features[0]
    xp = jnp.pad(x, ((0, 0), (1, 1), (1, 1)))
    pat = jnp.stack(
        [xp[:, ky : ky + H, kx : kx + W] for ky in range(3) for kx in range(3)],
        axis=1,
    ).reshape(256 * 9, H * W)
    pat = jnp.pad(pat, ((0, 0), (0, HWP - H * W)))

    w1 = conv_w.reshape(256, 256 * 9)
    w2 = jnp.concatenate(
        [
            cls_w.reshape(NA, 256),
            bbox_w.reshape(4 * NA, 256),
            jnp.zeros((128 - 5 * NA, 256), jnp.float32),
        ],
        axis=0,
    )
    b2 = jnp.concatenate(
        [cls_b, bbox_b, jnp.zeros((128 - 5 * NA,), jnp.float32)], axis=0
    )

    head_out = pl.pallas_call(
        _head_body,
        grid=(N_COL_T,),
        in_specs=[
            pl.BlockSpec((256 * 9, COL_T), lambda j: (0, j)),
            pl.BlockSpec((256, 256 * 9), lambda j: (0, 0)),
            pl.BlockSpec((256, 1), lambda j: (0, 0)),
            pl.BlockSpec((128, 256), lambda j: (0, 0)),
            pl.BlockSpec((128, 1), lambda j: (0, 0)),
        ],
        out_specs=pl.BlockSpec((128, COL_T), lambda j: (0, j)),
        out_shape=jax.ShapeDtypeStruct((128, HWP), jnp.float32),
    )(pat, w1, conv_b[:, None], w2, b2[:, None])

    head_out = head_out[:, : H * W]
    obj = head_out[:NA].T.reshape(-1)
    reg = head_out[NA : 5 * NA].T.reshape(-1, 4)

    topv, topi = lax.top_k(obj, PRE_NMS)
    deltas = reg[topi]
    anc = _ANCHORS[topi]

    widths = anc[:, 2] - anc[:, 0] + 1.0
    heights = anc[:, 3] - anc[:, 1] + 1.0
    cx = anc[:, 0] + 0.5 * widths
    cy = anc[:, 1] + 0.5 * heights
    dw = jnp.minimum(deltas[:, 2], BBOX_CLIP)
    dh = jnp.minimum(deltas[:, 3], BBOX_CLIP)
    pcx = deltas[:, 0] * widths + cx
    pcy = deltas[:, 1] * heights + cy
    pw = jnp.exp(dw) * widths
    ph = jnp.exp(dh) * heights
    x1 = jnp.clip(pcx - 0.5 * pw, 0.0, IM_W - 1.0)
    y1 = jnp.clip(pcy - 0.5 * ph, 0.0, IM_H - 1.0)
    x2 = jnp.clip(pcx + 0.5 * pw - 1.0, 0.0, IM_W - 1.0)
    y2 = jnp.clip(pcy + 0.5 * ph - 1.0, 0.0, IM_H - 1.0)
    boxes = jnp.stack([x1, y1, x2, y2], axis=1)
    area = (x2 - x1 + 1.0) * (y2 - y1 + 1.0)

    c5 = jnp.pad(
        jnp.stack([x1, y1, x2, y2, area], axis=0), ((0, 0), (0, NPAD - PRE_NMS))
    ).reshape(5, NR, NC)
    rows = jnp.pad(
        jnp.stack([x1, y1, x2, y2, area], axis=1),
        ((0, NPAD - PRE_NMS), (0, 3)),
    )

    keep = pl.pallas_call(
        _nms_body,
        in_specs=[
            pl.BlockSpec((5, NR, NC), lambda: (0, 0, 0)),
            pl.BlockSpec((NPAD, 8), lambda: (0, 0)),
        ],
        out_specs=pl.BlockSpec((NR, NC), lambda: (0, 0)),
        out_shape=jax.ShapeDtypeStruct((NR, NC), jnp.float32),
    )(c5, rows)

    keep = keep.reshape(NPAD)[:PRE_NMS] > 0.5
    scores = jax.nn.sigmoid(topv)
    masked = jnp.where(keep, scores, -1.0)
    selv, seli = lax.top_k(masked, POST_NMS)
    return jnp.concatenate([boxes[seli], selv[:, None]], axis=1)


# speedup vs baseline: 17.2307x; 1.0046x over previous
import numpy as np
import jax
import jax.numpy as jnp
from jax import lax
from jax.experimental import pallas as pl

H, W = 100, 152
STRIDE = 16
SIZES = (32.0, 64.0, 128.0, 256.0, 512.0)
RATIOS = (0.5, 1.0, 2.0)
NA = len(SIZES) * len(RATIOS)
PRE_NMS = 6000
POST_NMS = 1000
NMS_THRESH = 0.7
BBOX_CLIP = float(np.log(1000.0 / 16.0))
IM_W = W * STRIDE
IM_H = H * STRIDE

HWP = 15360
COL_T = 1024
N_COL_T = HWP // COL_T
NPAD = 6144
NR, NC = 48, 128


def _anchors_np():
    base = []
    for size in SIZES:
        area = size * size
        for r in RATIOS:
            w = np.sqrt(area / r)
            h = w * r
            base.append([-w / 2.0, -h / 2.0, w / 2.0, h / 2.0])
    base = np.asarray(base, np.float32)
    sx = np.arange(W, dtype=np.float32) * STRIDE
    sy = np.arange(H, dtype=np.float32) * STRIDE
    yy, xx = np.meshgrid(sy, sx, indexing="ij")
    shifts = np.stack([xx, yy, xx, yy], axis=-1).reshape(-1, 1, 4)
    return (shifts + base[None]).reshape(-1, 4).astype(np.float32)


_ANCHORS = jnp.asarray(_anchors_np())


def _head_body(p_ref, w1_ref, b1_ref, w2_ref, b2_ref, o_ref):
    t = jnp.dot(w1_ref[...], p_ref[...], preferred_element_type=jnp.float32)
    t = jnp.maximum(t + b1_ref[...], 0.0)
    o_ref[...] = (
        jnp.dot(w2_ref[...], t, preferred_element_type=jnp.float32) + b2_ref[...]
    )


def _nms_body(c_ref, rows_ref, keep_ref):
    x1, y1, x2, y2, area = c_ref[0], c_ref[1], c_ref[2], c_ref[3], c_ref[4]
    rid = lax.broadcasted_iota(jnp.int32, (NR, NC), 0)
    cid = lax.broadcasted_iota(jnp.int32, (NR, NC), 1)
    ii = rid * NC + cid

    def cond_fn(carry):
        i, kept, _ = carry
        return (i < PRE_NMS) & (kept < POST_NMS)

    def body_fn(carry):
        i, kept, sup = carry
        eq = ii == i
        s_i = jnp.max(jnp.where(eq, sup, 0.0))

        def live(s):
            row = rows_ref[pl.ds(i, 1), :]
            bx1, by1 = row[:, 0:1], row[:, 1:2]
            bx2, by2 = row[:, 2:3], row[:, 3:4]
            ba = row[:, 4:5]
            iw = jnp.maximum(
                jnp.minimum(x2, bx2) - jnp.maximum(x1, bx1) + 1.0, 0.0
            )
            ih = jnp.maximum(
                jnp.minimum(y2, by2) - jnp.maximum(y1, by1) + 1.0, 0.0
            )
            inter = iw * ih
            iou = inter / (area + ba - inter)
            hit = jnp.where((iou > NMS_THRESH) & (ii > i), 1.0, 0.0)
            return jnp.maximum(s, hit)

        sup2 = lax.cond(s_i < 0.5, live, lambda s: s, sup)
        return i + 1, kept + (1 - s_i.astype(jnp.int32)), sup2

    _, _, sup = lax.while_loop(
        cond_fn, body_fn, (0, 0, jnp.zeros((NR, NC), jnp.float32))
    )
    keep_ref[...] = 1.0 - sup


def kernel(features, conv_w, conv_b, cls_w, cls_b, bbox_w, bbox_b):
    x = features[0]
    xp = jnp.pad(x, ((0, 0), (1, 1), (1, 1)))
    pat = jnp.stack(
        [xp[:, ky : ky + H, kx : kx + W] for ky in range(3) for kx in range(3)],
        axis=1,
    ).reshape(256 * 9, H * W)
    pat = jnp.pad(pat, ((0, 0), (0, HWP - H * W)))

    w1 = conv_w.reshape(256, 256 * 9)
    w2 = jnp.concatenate(
        [
            cls_w.reshape(NA, 256),
            bbox_w.reshape(4 * NA, 256),
            jnp.zeros((128 - 5 * NA, 256), jnp.float32),
        ],
        axis=0,
    )
    b2 = jnp.concatenate(
        [cls_b, bbox_b, jnp.zeros((128 - 5 * NA,), jnp.float32)], axis=0
    )

    head_out = pl.pallas_call(
        _head_body,
        grid=(N_COL_T,),
        in_specs=[
            pl.BlockSpec((256 * 9, COL_T), lambda j: (0, j)),
            pl.BlockSpec((256, 256 * 9), lambda j: (0, 0)),
            pl.BlockSpec((256, 1), lambda j: (0, 0)),
            pl.BlockSpec((128, 256), lambda j: (0, 0)),
            pl.BlockSpec((128, 1), lambda j: (0, 0)),
        ],
        out_specs=pl.BlockSpec((128, COL_T), lambda j: (0, j)),
        out_shape=jax.ShapeDtypeStruct((128, HWP), jnp.float32),
    )(pat, w1, conv_b[:, None], w2, b2[:, None])

    head_out = head_out[:, : H * W]
    obj = head_out[:NA].T.reshape(-1)
    reg = head_out[NA : 5 * NA].T.reshape(-1, 4)

    topv, topi = lax.top_k(obj, PRE_NMS)
    deltas = reg[topi]
    anc = _ANCHORS[topi]

    widths = anc[:, 2] - anc[:, 0] + 1.0
    heights = anc[:, 3] - anc[:, 1] + 1.0
    cx = anc[:, 0] + 0.5 * widths
    cy = anc[:, 1] + 0.5 * heights
    dw = jnp.minimum(deltas[:, 2], BBOX_CLIP)
    dh = jnp.minimum(deltas[:, 3], BBOX_CLIP)
    pcx = deltas[:, 0] * widths + cx
    pcy = deltas[:, 1] * heights + cy
    pw = jnp.exp(dw) * widths
    ph = jnp.exp(dh) * heights
    x1 = jnp.clip(pcx - 0.5 * pw, 0.0, IM_W - 1.0)
    y1 = jnp.clip(pcy - 0.5 * ph, 0.0, IM_H - 1.0)
    x2 = jnp.clip(pcx + 0.5 * pw - 1.0, 0.0, IM_W - 1.0)
    y2 = jnp.clip(pcy + 0.5 * ph - 1.0, 0.0, IM_H - 1.0)
    boxes = jnp.stack([x1, y1, x2, y2], axis=1)
    area = (x2 - x1 + 1.0) * (y2 - y1 + 1.0)

    c5 = jnp.pad(
        jnp.stack([x1, y1, x2, y2, area], axis=0), ((0, 0), (0, NPAD - PRE_NMS))
    ).reshape(5, NR, NC)
    rows = jnp.pad(
        jnp.stack([x1, y1, x2, y2, area], axis=1),
        ((0, NPAD - PRE_NMS), (0, 3)),
    )

    keep = pl.pallas_call(
        _nms_body,
        in_specs=[
            pl.BlockSpec((5, NR, NC), lambda: (0, 0, 0)),
            pl.BlockSpec((NPAD, 8), lambda: (0, 0)),
        ],
        out_specs=pl.BlockSpec((NR, NC), lambda: (0, 0)),
        out_shape=jax.ShapeDtypeStruct((NR, NC), jnp.float32),
    )(c5, rows)

    keep = keep.reshape(NPAD)[:PRE_NMS] > 0.5
    scores = jax.nn.sigmoid(topv)
    masked = jnp.where(keep, scores, -1.0)
    selv, seli = lax.top_k(masked, POST_NMS)
    return jnp.concatenate([boxes[seli], selv[:, None]], axis=1)
